# baseline (speedup 1.0000x reference)
import functools
import jax
import jax.numpy as jnp
from jax import lax
from jax.experimental import pallas as pl
from jax.experimental.pallas import tpu as pltpu
from jax.experimental.pallas import tpu_sc as plsc

_N = 2408448
_NW = 32
_PW = _N // _NW
_CHE = 9408
_NCH = _PW // _CHE
_CHW = _CHE // 4
_TW = 15

_mesh = plsc.VectorSubcoreMesh(core_axis_name="c", subcore_axis_name="s")


@functools.partial(
    pl.kernel,
    mesh=_mesh,
    compiler_params=pltpu.CompilerParams(needs_layout_passes=False),
    out_type=(
        jax.ShapeDtypeStruct((_NW * 16,), jnp.float32),
        jax.ShapeDtypeStruct((_NW * 16,), jnp.float32),
    ),
    scratch_types=[
        pltpu.VMEM((_PW,), jnp.float32),
        pltpu.VMEM((16,), jnp.float32),
        pltpu.VMEM((16,), jnp.float32),
    ],
)
def _sc_reduce(img_hbm, mins_hbm, maxs_hbm, buf, mnv, mxv):
    wid = lax.axis_index("s") * 2 + lax.axis_index("c")
    pltpu.sync_copy(img_hbm.at[pl.ds(wid * _PW, _PW)], buf)
    inf = jnp.float32(jnp.inf)

    def body(i, carry):
        mn, mx = carry
        v = buf[pl.ds(i * 16, 16)]
        vm = jnp.where(v < 0.0, inf, v)
        return jnp.minimum(mn, vm), jnp.maximum(mx, v)

    mn0 = jnp.full((16,), inf, jnp.float32)
    mx0 = jnp.full((16,), -inf, jnp.float32)
    mn, mx = lax.fori_loop(0, _PW // 16, body, (mn0, mx0))
    mnv[...] = mn
    mxv[...] = mx
    pltpu.sync_copy(mnv, mins_hbm.at[pl.ds(wid * 16, 16)])
    pltpu.sync_copy(mxv, maxs_hbm.at[pl.ds(wid * 16, 16)])


@functools.partial(
    pl.kernel,
    mesh=_mesh,
    compiler_params=pltpu.CompilerParams(needs_layout_passes=False),
    out_type=jax.ShapeDtypeStruct((_N,), jnp.int32),
    scratch_types=[
        pltpu.VMEM((_CHE,), jnp.float32),
        pltpu.VMEM((_CHE,), jnp.int32),
        pltpu.VMEM((_NW * 16,), jnp.float32),
        pltpu.VMEM((_NW * 16,), jnp.float32),
    ],
)
def _sc_expand(img_hbm, mins_hbm, maxs_hbm, out_hbm, inbuf, wbuf, mnb, mxb):
    wid = lax.axis_index("s") * 2 + lax.axis_index("c")
    pltpu.sync_copy(mins_hbm, mnb)
    pltpu.sync_copy(maxs_hbm, mxb)

    inf = jnp.float32(jnp.inf)
    mn = jnp.full((16,), inf, jnp.float32)
    mx = jnp.full((16,), -inf, jnp.float32)
    for i in range(_NW):
        mn = jnp.minimum(mn, mnb[pl.ds(i * 16, 16)])
        mx = jnp.maximum(mx, mxb[pl.ds(i * 16, 16)])
    mmin = mn[0]
    gmax = mx[0]
    for k in range(1, 16):
        mmin = jnp.minimum(mmin, mn[k])
        gmax = jnp.maximum(gmax, mx[k])

    nab = mmin < inf
    img_min = jnp.where(nab, mmin, jnp.float32(0.0))
    imax = gmax - img_min
    ones = jnp.ones((16,), jnp.float32)
    denv = jnp.full((16,), 1.0, jnp.float32) - jnp.full((16,), img_min,
                                                        jnp.float32)
    imaxv = jnp.full((16,), imax, jnp.float32)
    rec_den = (ones / denv)[0]
    rec_imax = (ones / imaxv)[0]
    mf = jnp.where(nab, rec_den, jnp.float32(1.0))
    mf = jnp.where(imax != 0.0, rec_imax, mf)

    for c in range(_NCH):
        ebase = wid * _PW + c * _CHE
        pltpu.sync_copy(img_hbm.at[pl.ds(ebase, _CHE)], inbuf)

        def ebody(i, carry):
            v = inbuf[pl.ds(i * 16, 16)]
            scaled = (v - img_min) * mf
            y = scaled * jnp.float32(_TW - 1.0)
            t = y.astype(jnp.int32)
            ci = jnp.where(t.astype(jnp.float32) < y, t + 1, t)
            idx = ci + 1
            idx = jnp.where(v < 0.0, 0, idx)
            ok = (idx >= 1) & (idx <= _TW)
            sh = jnp.where(ok, _TW - idx, 0)
            word = jnp.where(ok, lax.shift_left(jnp.int32(1), sh), 0)
            wbuf[pl.ds(i * 16, 16)] = word
            return carry

        lax.fori_loop(0, _CHE // 16, ebody, 0)
        pltpu.sync_copy(wbuf, out_hbm.at[pl.ds(ebase, _CHE)])


def kernel(img):
    flat = img.reshape(_N)
    mins, maxs = _sc_reduce(flat)
    words = _sc_expand(flat, mins, maxs)
    w4 = words.reshape(16, 3, 224, 224)
    masks = (jnp.int32(1) << jnp.arange(_TW, dtype=jnp.int32)).reshape(
        _TW, 1, 1, 1, 1
    )
    return (w4[None] & masks) != 0


# speedup vs baseline: 1.2444x; 1.2444x over previous
import functools
import jax
import jax.numpy as jnp
from jax import lax
from jax.experimental import pallas as pl
from jax.experimental.pallas import tpu as pltpu
from jax.experimental.pallas import tpu_sc as plsc

_TW = 15
_B = 16
_CH = 3
_H = 224
_W = 224
_N = _B * _CH * _H * _W
_NW = 32
_PW = _N // _NW

_mesh = plsc.VectorSubcoreMesh(core_axis_name="c", subcore_axis_name="s")


@functools.partial(
    pl.kernel,
    mesh=_mesh,
    compiler_params=pltpu.CompilerParams(needs_layout_passes=False),
    out_type=(
        jax.ShapeDtypeStruct((_NW * 16,), jnp.float32),
        jax.ShapeDtypeStruct((_NW * 16,), jnp.float32),
    ),
    scratch_types=[
        pltpu.VMEM((_PW,), jnp.float32),
        pltpu.VMEM((16,), jnp.float32),
        pltpu.VMEM((16,), jnp.float32),
    ],
)
def _sc_reduce(img_hbm, mins_hbm, maxs_hbm, buf, mnv, mxv):
    wid = lax.axis_index("s") * 2 + lax.axis_index("c")
    pltpu.sync_copy(img_hbm.at[pl.ds(wid * _PW, _PW)], buf)
    inf = jnp.float32(jnp.inf)

    def body(i, carry):
        mn, mx = carry
        v = buf[pl.ds(i * 16, 16)]
        vm = jnp.where(v < 0.0, inf, v)
        return jnp.minimum(mn, vm), jnp.maximum(mx, v)

    mn0 = jnp.full((16,), inf, jnp.float32)
    mx0 = jnp.full((16,), -inf, jnp.float32)
    mn, mx = lax.fori_loop(0, _PW // 16, body, (mn0, mx0))
    mnv[...] = mn
    mxv[...] = mx
    pltpu.sync_copy(mnv, mins_hbm.at[pl.ds(wid * 16, 16)])
    pltpu.sync_copy(mxv, maxs_hbm.at[pl.ds(wid * 16, 16)])


def _pack_body(mn_ref, mx_ref, x_ref, o_ref):
    mmin = jnp.min(mn_ref[...])
    gmax = jnp.max(mx_ref[...])
    nab = mmin < jnp.inf
    img_min = jnp.where(nab, mmin, 0.0)
    mf = jnp.where(nab, 1.0 / (1.0 - img_min), 1.0)
    imax = gmax - img_min
    mf = jnp.where(imax != 0.0, 1.0 / imax, mf)

    x = x_ref[...]
    y = ((x - img_min) * mf) * jnp.float32(_TW - 1)
    idx = jnp.ceil(y).astype(jnp.int32) + 1
    idx = jnp.where(x < 0.0, 0, idx)
    ok = (idx >= 1) & (idx <= _TW)
    sh = jnp.where(ok, _TW - idx, 0)
    word = jnp.where(ok, jnp.left_shift(jnp.int32(1), sh), 0)
    o_ref[...] = word.astype(jnp.uint16)


def kernel(img):
    mins, maxs = _sc_reduce(img.reshape(_N))
    words = pl.pallas_call(
        _pack_body,
        grid=(_B,),
        in_specs=[
            pl.BlockSpec((_NW * 16,), lambda i: (0,)),
            pl.BlockSpec((_NW * 16,), lambda i: (0,)),
            pl.BlockSpec((1, _CH, _H, _W), lambda i: (i, 0, 0, 0)),
        ],
        out_specs=pl.BlockSpec((1, _CH, _H, _W), lambda i: (i, 0, 0, 0)),
        out_shape=jax.ShapeDtypeStruct((_B, _CH, _H, _W), jnp.uint16),
    )(mins, maxs, img)
    masks = (jnp.uint16(1) << jnp.arange(_TW, dtype=jnp.uint16)).reshape(
        _TW, 1, 1, 1, 1
    )
    return (words[None] & masks) != 0
